# Initial kernel scaffold; baseline (speedup 1.0000x reference)
import jax
import jax.numpy as jnp
from jax.experimental import pallas as pl


def kernel(x, W_enc, b_dec, W_dec):
    raise NotImplementedError("write your pallas kernel here")


# speedup vs baseline: 4.1141x; 4.1141x over previous
import jax
import jax.numpy as jnp
from jax.experimental import pallas as pl

_TOP_K = 32
_D_VIT = 768
_D_SAE = 12288


def _encode_body(x_ref, we_ref, bd_ref, acts_ref):
    xc = x_ref[...] - bd_ref[...]
    acts_ref[...] = jnp.maximum(
        jnp.dot(xc, we_ref[...], preferred_element_type=jnp.float32), 0.0
    )


def _decode_body(acts_ref, wd_ref, bd_ref, o_ref):
    acts = acts_ref[...]
    ai = jax.lax.bitcast_convert_type(acts, jnp.int32)
    tb = acts.shape[0]

    def level(i, t):
        cand = t | (jnp.int32(1) << (30 - i))
        cnt = jnp.sum((ai >= cand).astype(jnp.int32), axis=1, keepdims=True)
        return jnp.where(cnt >= _TOP_K, cand, t)

    t = jax.lax.fori_loop(0, 31, level, jnp.zeros((tb, 1), jnp.int32))
    feats = jnp.where(ai >= t, acts, 0.0)
    o_ref[...] = (
        jnp.dot(feats, wd_ref[...], preferred_element_type=jnp.float32)
        + bd_ref[...]
    )


def kernel(x, W_enc, b_dec, W_dec):
    B, S, DV = x.shape
    n = B * S
    x2 = x.reshape(n, DV)
    bd2 = b_dec.reshape(1, DV)

    ta = 128 if n % 128 == 0 else n
    acts = pl.pallas_call(
        _encode_body,
        grid=(n // ta,),
        in_specs=[
            pl.BlockSpec((ta, DV), lambda i: (i, 0)),
            pl.BlockSpec((DV, _D_SAE), lambda i: (0, 0)),
            pl.BlockSpec((1, DV), lambda i: (0, 0)),
        ],
        out_specs=pl.BlockSpec((ta, _D_SAE), lambda i: (i, 0)),
        out_shape=jax.ShapeDtypeStruct((n, _D_SAE), jnp.float32),
    )(x2, W_enc, bd2)

    tb = 64 if n % 64 == 0 else n
    out = pl.pallas_call(
        _decode_body,
        grid=(n // tb,),
        in_specs=[
            pl.BlockSpec((tb, _D_SAE), lambda i: (i, 0)),
            pl.BlockSpec((_D_SAE, DV), lambda i: (0, 0)),
            pl.BlockSpec((1, DV), lambda i: (0, 0)),
        ],
        out_specs=pl.BlockSpec((tb, DV), lambda i: (i, 0)),
        out_shape=jax.ShapeDtypeStruct((n, DV), jnp.float32),
    )(acts, W_dec, bd2)

    return out.reshape(B, S, DV)
